# baseline (speedup 1.0000x reference)
import jax
import jax.numpy as jnp
from jax.experimental import pallas as pl
from jax.experimental.pallas import tpu as pltpu

_B = 8
_ROWS = 80
_W = 65536
_BLOCK_W = 4096
_GRID = (_B, _W // _BLOCK_W)
_LOG2E = 1.4426950408889634
_LN2 = 0.6931471805599453
_SCALE = 0.25 * _LN2 / float(_B * _ROWS * _W)


def _focal_body(x_ref, t_ref, o_ref):
    f1 = jnp.bfloat16(1.0)
    x = x_ref[...]
    t = t_ref[...].astype(jnp.bfloat16)
    z = x * _LOG2E
    e2 = jnp.exp2(z)
    u = 1.0 + e2
    lg = jnp.log2(u).astype(jnp.bfloat16)
    r = (1.0 / u).astype(jnp.bfloat16)
    zb = z.astype(jnp.bfloat16)
    ce2 = lg - zb * t
    k = f1 - (t + t)
    w = (f1 - t) - r * k
    s = jnp.sum((k + jnp.bfloat16(2.0)) * ce2 * (w * w),
                dtype=jnp.float32)

    @pl.when((pl.program_id(0) == 0) & (pl.program_id(1) == 0))
    def _init():
        o_ref[0, 0] = 0.0

    o_ref[0, 0] += s * _SCALE


def kernel(logits, targets):
    x = jnp.transpose(logits, (0, 2, 1))
    t = jnp.transpose(targets, (0, 2, 1))
    out = pl.pallas_call(
        _focal_body,
        grid=_GRID,
        in_specs=[
            pl.BlockSpec((1, _ROWS, _BLOCK_W), lambda b, i: (b, 0, i)),
            pl.BlockSpec((1, _ROWS, _BLOCK_W), lambda b, i: (b, 0, i)),
        ],
        out_specs=pl.BlockSpec(memory_space=pltpu.SMEM),
        out_shape=jax.ShapeDtypeStruct((1, 1), jnp.float32),
    )(x, t)
    return out[0, 0]


# speedup vs baseline: 1.0954x; 1.0954x over previous
import jax
import jax.numpy as jnp
from jax.experimental import pallas as pl
from jax.experimental.pallas import tpu as pltpu

_B = 8
_ROWS = 80
_W = 65536
_BLOCK_W = 8192
_GRID = (_B, _W // _BLOCK_W)
_LOG2E = 1.4426950408889634
_LN2 = 0.6931471805599453
_SCALE = 0.25 * _LN2 / float(_B * _ROWS * _W)


def _focal_body(x_ref, t_ref, o_ref):
    f1 = jnp.bfloat16(1.0)
    x = x_ref[...]
    t = t_ref[...].astype(jnp.bfloat16)
    z = x * _LOG2E
    e2 = jnp.exp2(z)
    u = 1.0 + e2
    lg = jnp.log2(u).astype(jnp.bfloat16)
    r = (1.0 / u).astype(jnp.bfloat16)
    zb = z.astype(jnp.bfloat16)
    ce2 = lg - zb * t
    k = f1 - (t + t)
    w = (f1 - t) - r * k
    s = jnp.sum((k + jnp.bfloat16(2.0)) * ce2 * (w * w),
                dtype=jnp.float32)

    @pl.when((pl.program_id(0) == 0) & (pl.program_id(1) == 0))
    def _init():
        o_ref[0, 0] = 0.0

    o_ref[0, 0] += s * _SCALE


def kernel(logits, targets):
    x = jnp.transpose(logits, (0, 2, 1))
    t = jnp.transpose(targets, (0, 2, 1))
    out = pl.pallas_call(
        _focal_body,
        grid=_GRID,
        in_specs=[
            pl.BlockSpec((1, _ROWS, _BLOCK_W), lambda b, i: (b, 0, i)),
            pl.BlockSpec((1, _ROWS, _BLOCK_W), lambda b, i: (b, 0, i)),
        ],
        out_specs=pl.BlockSpec(memory_space=pltpu.SMEM),
        out_shape=jax.ShapeDtypeStruct((1, 1), jnp.float32),
    )(x, t)
    return out[0, 0]
